# Initial kernel scaffold; baseline (speedup 1.0000x reference)
import jax
import jax.numpy as jnp
from jax.experimental import pallas as pl


def kernel(x, emb_table, W0, b0, P0, Wp0, bp0, W1, b1, P1, Wp1, bp1, W_pen, b_pen, W_out, b_out):
    raise NotImplementedError("write your pallas kernel here")


# speedup vs baseline: 1.5377x; 1.5377x over previous
import functools

import numpy as np
import jax
import jax.numpy as jnp
from jax import lax
from jax.experimental import pallas as pl
from jax.experimental.pallas import tpu as pltpu
from jax.experimental.pallas import tpu_sc as plsc

_NW = 32


def _sc_gather(x_flat, table, bsz, seq):
    n = x_flat.shape[0]
    v, e = table.shape
    bpw = n // _NW
    rpw = bsz // _NW
    mesh = plsc.VectorSubcoreMesh(core_axis_name="c", subcore_axis_name="s")

    def body(idx_hbm, table_hbm, out_hbm, idx_v, rows_v, sem):
        sid = lax.axis_index("s")
        wid = sid * 2 + lax.axis_index("c")
        base = wid * bpw
        pltpu.sync_copy(idx_hbm.at[pl.ds(base, bpw)], idx_v)
        pltpu.async_copy(table_hbm.at[idx_v], rows_v, sem).wait()
        pltpu.sync_copy(rows_v, out_hbm.at[pl.ds(base, bpw)])

    return pl.kernel(
        body,
        mesh=mesh,
        out_type=jax.ShapeDtypeStruct((n, e), jnp.float32),
        scratch_types=[
            pltpu.VMEM((bpw,), jnp.int32),
            pltpu.VMEM((bpw, e), jnp.float32),
            pltpu.SemaphoreType.DMA,
        ],
        compiler_params=pltpu.CompilerParams(use_tc_tiling_on_sc=False),
    )(x_flat, table)


_INV_SQRT_P0 = float(1.0 / np.sqrt(64.0))
_INV_SQRT_P1 = float(1.0 / np.sqrt(32.0))


def _mlp_body(emb_ref, W0_ref, b0_ref, P0_ref, P0T_ref, Wp0_ref, bp0_ref,
              W1_ref, b1_ref, P1_ref, P1T_ref, Wp1_ref, bp1_ref,
              Wpen_ref, bpen_ref, pen_ref, pe_ref):
    f32 = jnp.float32
    h = emb_ref[...]
    z0 = jnp.dot(h, W0_ref[...], preferred_element_type=f32) + b0_ref[...]
    a0 = jnp.maximum(z0, 0.0)
    s0 = jnp.dot(a0, P0T_ref[...], preferred_element_type=f32) * _INV_SQRT_P0
    e0 = jnp.exp(s0 - jnp.max(s0, axis=-1, keepdims=True))
    attn0 = e0 / jnp.sum(e0, axis=-1, keepdims=True)
    recon0 = jnp.dot(attn0, P0_ref[...], preferred_element_type=f32)
    pred0 = jnp.dot(a0, Wp0_ref[...], preferred_element_type=f32) + bp0_ref[...]

    z1 = jnp.dot(recon0, W1_ref[...], preferred_element_type=f32) + b1_ref[...]
    a1 = jnp.maximum(z1, 0.0)
    s1 = jnp.dot(a1, P1T_ref[...], preferred_element_type=f32) * _INV_SQRT_P1
    e1 = jnp.exp(s1 - jnp.max(s1, axis=-1, keepdims=True))
    attn1 = e1 / jnp.sum(e1, axis=-1, keepdims=True)
    recon1 = jnp.dot(attn1, P1_ref[...], preferred_element_type=f32)
    pred1 = jnp.dot(a1, Wp1_ref[...], preferred_element_type=f32) + bp1_ref[...]

    pen = jnp.maximum(
        jnp.dot(recon1, Wpen_ref[...], preferred_element_type=f32) + bpen_ref[...], 0.0)
    pen_ref[...] = pen

    err0 = jnp.mean((pred0 - pen) ** 2, axis=-1, keepdims=True)
    err1 = jnp.mean((pred1 - pen) ** 2, axis=-1, keepdims=True)
    pe_ref[...] = jnp.concatenate(
        [err0, err1, jnp.zeros((err0.shape[0], 6), f32)], axis=1)


def _mlp(embf, W0, b0, P0, Wp0, bp0, W1, b1, P1, Wp1, bp1, W_pen, b_pen):
    bsz = embf.shape[0]
    bb = 256
    grid = bsz // bb
    full = lambda i: (0, 0)
    return pl.pallas_call(
        _mlp_body,
        grid=(grid,),
        in_specs=[
            pl.BlockSpec((bb, embf.shape[1]), lambda i: (i, 0)),
            pl.BlockSpec(W0.shape, full),
            pl.BlockSpec((1, 64), full),
            pl.BlockSpec(P0.shape, full),
            pl.BlockSpec((64, 8), full),
            pl.BlockSpec(Wp0.shape, full),
            pl.BlockSpec((1, 32), full),
            pl.BlockSpec(W1.shape, full),
            pl.BlockSpec((1, 32), full),
            pl.BlockSpec(P1.shape, full),
            pl.BlockSpec((32, 8), full),
            pl.BlockSpec(Wp1.shape, full),
            pl.BlockSpec((1, 32), full),
            pl.BlockSpec(W_pen.shape, full),
            pl.BlockSpec((1, 32), full),
        ],
        out_specs=[
            pl.BlockSpec((bb, 32), lambda i: (i, 0)),
            pl.BlockSpec((bb, 8), lambda i: (i, 0)),
        ],
        out_shape=[
            jax.ShapeDtypeStruct((bsz, 32), jnp.float32),
            jax.ShapeDtypeStruct((bsz, 8), jnp.float32),
        ],
    )(embf, W0, b0.reshape(1, -1), P0, P0.T, Wp0, bp0.reshape(1, -1),
      W1, b1.reshape(1, -1), P1, P1.T, Wp1, bp1.reshape(1, -1),
      W_pen, b_pen.reshape(1, -1))


def _proj_body(pen_ref, w_ref, b_ref, o_ref):
    o_ref[...] = jnp.dot(pen_ref[...], w_ref[...],
                         preferred_element_type=jnp.float32) + b_ref[...]


def _proj(pen, W_out, b_out):
    bsz, k = pen.shape
    v = W_out.shape[1]
    bn = 2048
    return pl.pallas_call(
        _proj_body,
        grid=(pl.cdiv(v, bn),),
        in_specs=[
            pl.BlockSpec((bsz, k), lambda i: (0, 0)),
            pl.BlockSpec((k, bn), lambda i: (0, i)),
            pl.BlockSpec((1, bn), lambda i: (0, i)),
        ],
        out_specs=pl.BlockSpec((bsz, bn), lambda i: (0, i)),
        out_shape=jax.ShapeDtypeStruct((bsz, v), jnp.float32),
    )(pen, W_out, b_out.reshape(1, -1))


def kernel(x, emb_table, W0, b0, P0, Wp0, bp0, W1, b1, P1, Wp1, bp1,
           W_pen, b_pen, W_out, b_out):
    bsz, seq = x.shape
    e = emb_table.shape[1]
    x_flat = x.reshape(bsz * seq).astype(jnp.int32)
    rows = _sc_gather(x_flat, emb_table, bsz, seq)
    embf = rows.reshape(bsz, seq * e)
    pen, pe = _mlp(embf, W0, b0, P0, Wp0, bp0, W1, b1, P1, Wp1, bp1,
                   W_pen, b_pen)
    output = _proj(pen, W_out, b_out)
    pred_errors = pe[:, :2].T
    return (output, pred_errors)


# speedup vs baseline: 1.5409x; 1.0021x over previous
import functools

import numpy as np
import jax
import jax.numpy as jnp
from jax import lax
from jax.experimental import pallas as pl
from jax.experimental.pallas import tpu as pltpu
from jax.experimental.pallas import tpu_sc as plsc

_NW = 32


def _sc_gather(x_flat, table, bsz, seq):
    n = x_flat.shape[0]
    v, e = table.shape
    bpw = n // _NW
    rpw = bsz // _NW
    mesh = plsc.VectorSubcoreMesh(core_axis_name="c", subcore_axis_name="s")

    def body(idx_hbm, table_hbm, out_hbm, idx_v, rows_v, sem):
        sid = lax.axis_index("s")
        wid = sid * 2 + lax.axis_index("c")
        base = wid * bpw
        pltpu.sync_copy(idx_hbm.at[pl.ds(base, bpw)], idx_v)
        pltpu.async_copy(table_hbm.at[idx_v], rows_v, sem).wait()
        pltpu.sync_copy(rows_v, out_hbm.at[pl.ds(base, bpw)])

    return pl.kernel(
        body,
        mesh=mesh,
        out_type=jax.ShapeDtypeStruct((n, e), jnp.float32),
        scratch_types=[
            pltpu.VMEM((bpw,), jnp.int32),
            pltpu.VMEM((bpw, e), jnp.float32),
            pltpu.SemaphoreType.DMA,
        ],
        compiler_params=pltpu.CompilerParams(use_tc_tiling_on_sc=False),
    )(x_flat, table)


_INV_SQRT_P0 = float(1.0 / np.sqrt(64.0))
_INV_SQRT_P1 = float(1.0 / np.sqrt(32.0))


def _mlp_body(emb_ref, W0_ref, b0_ref, P0_ref, P0T_ref, Wp0_ref, bp0_ref,
              W1_ref, b1_ref, P1_ref, P1T_ref, Wp1_ref, bp1_ref,
              Wpen_ref, bpen_ref, pen_ref, pe_ref):
    f32 = jnp.float32
    h = emb_ref[...]
    z0 = jnp.dot(h, W0_ref[...], preferred_element_type=f32) + b0_ref[...]
    a0 = jnp.maximum(z0, 0.0)
    s0 = jnp.dot(a0, P0T_ref[...], preferred_element_type=f32) * _INV_SQRT_P0
    e0 = jnp.exp(s0 - jnp.max(s0, axis=-1, keepdims=True))
    attn0 = e0 / jnp.sum(e0, axis=-1, keepdims=True)
    recon0 = jnp.dot(attn0, P0_ref[...], preferred_element_type=f32)
    pred0 = jnp.dot(a0, Wp0_ref[...], preferred_element_type=f32) + bp0_ref[...]

    z1 = jnp.dot(recon0, W1_ref[...], preferred_element_type=f32) + b1_ref[...]
    a1 = jnp.maximum(z1, 0.0)
    s1 = jnp.dot(a1, P1T_ref[...], preferred_element_type=f32) * _INV_SQRT_P1
    e1 = jnp.exp(s1 - jnp.max(s1, axis=-1, keepdims=True))
    attn1 = e1 / jnp.sum(e1, axis=-1, keepdims=True)
    recon1 = jnp.dot(attn1, P1_ref[...], preferred_element_type=f32)
    pred1 = jnp.dot(a1, Wp1_ref[...], preferred_element_type=f32) + bp1_ref[...]

    pen = jnp.maximum(
        jnp.dot(recon1, Wpen_ref[...], preferred_element_type=f32) + bpen_ref[...], 0.0)
    pen_ref[...] = pen

    err0 = jnp.mean((pred0 - pen) ** 2, axis=-1, keepdims=True)
    err1 = jnp.mean((pred1 - pen) ** 2, axis=-1, keepdims=True)
    pe_ref[...] = jnp.concatenate(
        [err0, err1, jnp.zeros((err0.shape[0], 6), f32)], axis=1)


def _mlp(embf, W0, b0, P0, Wp0, bp0, W1, b1, P1, Wp1, bp1, W_pen, b_pen):
    bsz = embf.shape[0]
    bb = 256
    grid = bsz // bb
    full = lambda i: (0, 0)
    return pl.pallas_call(
        _mlp_body,
        grid=(grid,),
        in_specs=[
            pl.BlockSpec((bb, embf.shape[1]), lambda i: (i, 0)),
            pl.BlockSpec(W0.shape, full),
            pl.BlockSpec((1, 64), full),
            pl.BlockSpec(P0.shape, full),
            pl.BlockSpec((64, 8), full),
            pl.BlockSpec(Wp0.shape, full),
            pl.BlockSpec((1, 32), full),
            pl.BlockSpec(W1.shape, full),
            pl.BlockSpec((1, 32), full),
            pl.BlockSpec(P1.shape, full),
            pl.BlockSpec((32, 8), full),
            pl.BlockSpec(Wp1.shape, full),
            pl.BlockSpec((1, 32), full),
            pl.BlockSpec(W_pen.shape, full),
            pl.BlockSpec((1, 32), full),
        ],
        out_specs=[
            pl.BlockSpec((bb, 32), lambda i: (i, 0)),
            pl.BlockSpec((bb, 8), lambda i: (i, 0)),
        ],
        out_shape=[
            jax.ShapeDtypeStruct((bsz, 32), jnp.float32),
            jax.ShapeDtypeStruct((bsz, 8), jnp.float32),
        ],
    )(embf, W0, b0.reshape(1, -1), P0, P0.T, Wp0, bp0.reshape(1, -1),
      W1, b1.reshape(1, -1), P1, P1.T, Wp1, bp1.reshape(1, -1),
      W_pen, b_pen.reshape(1, -1))


def _proj_body(pen_ref, w_ref, b_ref, o_ref):
    o_ref[...] = jnp.dot(pen_ref[...], w_ref[...],
                         preferred_element_type=jnp.float32) + b_ref[...]


def _proj(pen, W_out, b_out):
    bsz, k = pen.shape
    v = W_out.shape[1]
    bn = 4096
    return pl.pallas_call(
        _proj_body,
        grid=(pl.cdiv(v, bn),),
        in_specs=[
            pl.BlockSpec((bsz, k), lambda i: (0, 0)),
            pl.BlockSpec((k, bn), lambda i: (0, i)),
            pl.BlockSpec((1, bn), lambda i: (0, i)),
        ],
        out_specs=pl.BlockSpec((bsz, bn), lambda i: (0, i)),
        out_shape=jax.ShapeDtypeStruct((bsz, v), jnp.float32),
    )(pen, W_out, b_out.reshape(1, -1))


def kernel(x, emb_table, W0, b0, P0, Wp0, bp0, W1, b1, P1, Wp1, bp1,
           W_pen, b_pen, W_out, b_out):
    bsz, seq = x.shape
    e = emb_table.shape[1]
    x_flat = x.reshape(bsz * seq).astype(jnp.int32)
    rows = _sc_gather(x_flat, emb_table, bsz, seq)
    embf = rows.reshape(bsz, seq * e)
    pen, pe = _mlp(embf, W0, b0, P0, Wp0, bp0, W1, b1, P1, Wp1, bp1,
                   W_pen, b_pen)
    output = _proj(pen, W_out, b_out)
    pred_errors = pe[:, :2].T
    return (output, pred_errors)


# speedup vs baseline: 1.5792x; 1.0248x over previous
import functools

import numpy as np
import jax
import jax.numpy as jnp
from jax import lax
from jax.experimental import pallas as pl
from jax.experimental.pallas import tpu as pltpu
from jax.experimental.pallas import tpu_sc as plsc

_NW = 32


def _sc_gather(x_flat, table, bsz, seq):
    n = x_flat.shape[0]
    v, e = table.shape
    bpw = n // _NW
    rpw = bsz // _NW
    mesh = plsc.VectorSubcoreMesh(core_axis_name="c", subcore_axis_name="s")

    def body(idx_hbm, table_hbm, out_hbm, idx_v, rows_v, sem):
        sid = lax.axis_index("s")
        wid = sid * 2 + lax.axis_index("c")
        base = wid * bpw
        pltpu.sync_copy(idx_hbm.at[pl.ds(base, bpw)], idx_v)
        pltpu.async_copy(table_hbm.at[idx_v], rows_v, sem).wait()
        pltpu.sync_copy(rows_v, out_hbm.at[pl.ds(base, bpw)])

    return pl.kernel(
        body,
        mesh=mesh,
        out_type=jax.ShapeDtypeStruct((n, e), jnp.float32),
        scratch_types=[
            pltpu.VMEM((bpw,), jnp.int32),
            pltpu.VMEM((bpw, e), jnp.float32),
            pltpu.SemaphoreType.DMA,
        ],
        compiler_params=pltpu.CompilerParams(use_tc_tiling_on_sc=False),
    )(x_flat, table)


_INV_SQRT_P0 = float(1.0 / np.sqrt(64.0))
_INV_SQRT_P1 = float(1.0 / np.sqrt(32.0))


def _mlp_body(emb_ref, W0_ref, b0_ref, P0_ref, P0T_ref, Wp0_ref, bp0_ref,
              W1_ref, b1_ref, P1_ref, P1T_ref, Wp1_ref, bp1_ref,
              Wpen_ref, bpen_ref, pen_ref, pe_ref):
    f32 = jnp.float32
    bb = emb_ref.shape[0] // 25
    z0 = jnp.zeros((bb, W0_ref.shape[1]), f32) + b0_ref[...]
    for j in range(25):
        hj = emb_ref[pl.Slice(j, bb, 25), :]
        z0 = z0 + jnp.dot(hj, W0_ref[pl.ds(128 * j, 128), :],
                          preferred_element_type=f32)
    a0 = jnp.maximum(z0, 0.0)
    s0 = jnp.dot(a0, P0T_ref[...], preferred_element_type=f32) * _INV_SQRT_P0
    e0 = jnp.exp(s0 - jnp.max(s0, axis=-1, keepdims=True))
    attn0 = e0 / jnp.sum(e0, axis=-1, keepdims=True)
    recon0 = jnp.dot(attn0, P0_ref[...], preferred_element_type=f32)
    pred0 = jnp.dot(a0, Wp0_ref[...], preferred_element_type=f32) + bp0_ref[...]

    z1 = jnp.dot(recon0, W1_ref[...], preferred_element_type=f32) + b1_ref[...]
    a1 = jnp.maximum(z1, 0.0)
    s1 = jnp.dot(a1, P1T_ref[...], preferred_element_type=f32) * _INV_SQRT_P1
    e1 = jnp.exp(s1 - jnp.max(s1, axis=-1, keepdims=True))
    attn1 = e1 / jnp.sum(e1, axis=-1, keepdims=True)
    recon1 = jnp.dot(attn1, P1_ref[...], preferred_element_type=f32)
    pred1 = jnp.dot(a1, Wp1_ref[...], preferred_element_type=f32) + bp1_ref[...]

    pen = jnp.maximum(
        jnp.dot(recon1, Wpen_ref[...], preferred_element_type=f32) + bpen_ref[...], 0.0)
    pen_ref[...] = pen

    err0 = jnp.mean((pred0 - pen) ** 2, axis=-1, keepdims=True)
    err1 = jnp.mean((pred1 - pen) ** 2, axis=-1, keepdims=True)
    pe_ref[...] = jnp.concatenate(
        [err0, err1, jnp.zeros((err0.shape[0], 6), f32)], axis=1)


def _mlp(emb128, W0, b0, P0, Wp0, bp0, W1, b1, P1, Wp1, bp1, W_pen, b_pen):
    bsz = emb128.shape[0] * 128 // W0.shape[0]
    bb = 256
    grid = bsz // bb
    full = lambda i: (0, 0)
    return pl.pallas_call(
        _mlp_body,
        grid=(grid,),
        in_specs=[
            pl.BlockSpec((bb * 25, 128), lambda i: (i, 0)),
            pl.BlockSpec(W0.shape, full),
            pl.BlockSpec((1, 64), full),
            pl.BlockSpec(P0.shape, full),
            pl.BlockSpec((64, 8), full),
            pl.BlockSpec(Wp0.shape, full),
            pl.BlockSpec((1, 32), full),
            pl.BlockSpec(W1.shape, full),
            pl.BlockSpec((1, 32), full),
            pl.BlockSpec(P1.shape, full),
            pl.BlockSpec((32, 8), full),
            pl.BlockSpec(Wp1.shape, full),
            pl.BlockSpec((1, 32), full),
            pl.BlockSpec(W_pen.shape, full),
            pl.BlockSpec((1, 32), full),
        ],
        out_specs=[
            pl.BlockSpec((bb, 32), lambda i: (i, 0)),
            pl.BlockSpec((bb, 8), lambda i: (i, 0)),
        ],
        out_shape=[
            jax.ShapeDtypeStruct((bsz, 32), jnp.float32),
            jax.ShapeDtypeStruct((bsz, 8), jnp.float32),
        ],
    )(emb128, W0, b0.reshape(1, -1), P0, P0.T, Wp0, bp0.reshape(1, -1),
      W1, b1.reshape(1, -1), P1, P1.T, Wp1, bp1.reshape(1, -1),
      W_pen, b_pen.reshape(1, -1))


def _proj_body(pen_ref, w_ref, b_ref, o_ref):
    o_ref[...] = jnp.dot(pen_ref[...], w_ref[...],
                         preferred_element_type=jnp.float32) + b_ref[...]


def _proj(pen, W_out, b_out):
    bsz, k = pen.shape
    v = W_out.shape[1]
    bn = 4096
    return pl.pallas_call(
        _proj_body,
        grid=(pl.cdiv(v, bn),),
        in_specs=[
            pl.BlockSpec((bsz, k), lambda i: (0, 0)),
            pl.BlockSpec((k, bn), lambda i: (0, i)),
            pl.BlockSpec((1, bn), lambda i: (0, i)),
        ],
        out_specs=pl.BlockSpec((bsz, bn), lambda i: (0, i)),
        out_shape=jax.ShapeDtypeStruct((bsz, v), jnp.float32),
    )(pen, W_out, b_out.reshape(1, -1))


def kernel(x, emb_table, W0, b0, P0, Wp0, bp0, W1, b1, P1, Wp1, bp1,
           W_pen, b_pen, W_out, b_out):
    bsz, seq = x.shape
    e = emb_table.shape[1]
    x_flat = x.reshape(bsz * seq).astype(jnp.int32)
    rows = _sc_gather(x_flat, emb_table, bsz, seq)
    emb128 = rows.reshape(bsz * seq * e // 128, 128)
    pen, pe = _mlp(emb128, W0, b0, P0, Wp0, bp0, W1, b1, P1, Wp1, bp1,
                   W_pen, b_pen)
    output = _proj(pen, W_out, b_out)
    pred_errors = pe[:, :2].T
    return (output, pred_errors)
